# baseline (speedup 1.0000x reference)
import functools
import math

import jax
import jax.numpy as jnp
from jax import lax
from jax.experimental import pallas as pl
from jax.experimental.pallas import tpu as pltpu
from jax.experimental.pallas import tpu_sc as plsc

N = 10000
E = 320000
D = 128
H = 16
K = 32

NC = 2
NS = 16
NW = NC * NS

GPAD = 10240
EPAD = GPAD * K
GPW = GPAD // NW
CHUNK_G = 32
CHUNK_E = CHUNK_G * K
NCHUNK = GPW // CHUNK_G
BLK = CHUNK_E // 128
BPW = EPAD // 128 // NW
NBUF = 3

_INV_SCALE = 1.0 / math.sqrt(float(D))


def _nonneg(w):
    return jnp.where(w > 0, w + 1.0, jnp.exp(w))


def _emb_body(x_ref, w_ref, o_ref):
    wn = _nonneg(w_ref[...])
    o_ref[...] = lax.dot_general(
        x_ref[...], wn,
        dimension_numbers=(((1,), (1,)), ((), ())),
        preferred_element_type=jnp.float32,
    ) * _INV_SCALE


def _emb_call(x, w_cat):
    return pl.pallas_call(
        _emb_body,
        out_shape=jax.ShapeDtypeStruct((N, 2 * H), jnp.float32),
    )(x, w_cat)


@functools.lru_cache(maxsize=None)
def _make_sc_gather():
    mesh = plsc.VectorSubcoreMesh(
        core_axis_name="c", subcore_axis_name="s", num_cores=NC, num_subcores=NS)

    @functools.partial(
        pl.kernel,
        out_type=jax.ShapeDtypeStruct((GPAD, H), jnp.float32),
        mesh=mesh,
        compiler_params=pltpu.CompilerParams(use_tc_tiling_on_sc=False),
        scratch_types=[
            [pltpu.VMEM((CHUNK_E,), jnp.int32)] * NBUF,
            [pltpu.VMEM((CHUNK_E,), jnp.int32)] * NBUF,
            [pltpu.VMEM((CHUNK_E, H), jnp.float32)] * NBUF,
            [pltpu.VMEM((CHUNK_E, H), jnp.float32)] * NBUF,
            [pltpu.VMEM((CHUNK_G, H), jnp.float32)] * NBUF,
            [pltpu.SemaphoreType.DMA] * NBUF,
            [pltpu.SemaphoreType.DMA] * NBUF,
            [pltpu.SemaphoreType.DMA] * NBUF,
            [pltpu.SemaphoreType.DMA] * NBUF,
            pltpu.VMEM_SHARED((N, H), jnp.float32),
            pltpu.VMEM_SHARED((N, H), jnp.float32),
            pltpu.SemaphoreType.DMA,
        ],
    )
    def _sc_gather(q_hbm, k_hbm, a0_hbm, a1_hbm, out_hbm,
                   idx_k, idx_q, krows, qrows, outc, gsem, gsem2, isem, osem,
                   q_sp, k_sp, ssem):
        wid = lax.axis_index("s") * NC + lax.axis_index("c")
        sid = lax.axis_index("s")

        rows_per_tile = N // NS
        r0 = sid * rows_per_tile
        h1 = pltpu.async_copy(q_hbm.at[pl.ds(r0, rows_per_tile)],
                              q_sp.at[pl.ds(r0, rows_per_tile)], ssem)
        h1.wait()
        plsc.subcore_barrier()
        del k_sp

        def fire_idx(c, b):
            e0 = wid * (GPW * K) + c * CHUNK_E
            return [
                pltpu.async_copy(a0_hbm.at[pl.ds(e0, CHUNK_E)], idx_k[b], isem[b]),
                pltpu.async_copy(a1_hbm.at[pl.ds(e0, CHUNK_E)], idx_q[b], isem[b]),
            ]

        def fire_gather(c, b, idx_handles):
            for hh in idx_handles:
                hh.wait()
            return [
                pltpu.async_copy(k_hbm.at[idx_k[b]], krows[b], gsem2[b]),
                pltpu.async_copy(q_sp.at[idx_q[b]], qrows[b], gsem[b]),
            ]

        def compute(c, b):
            qr, kr, oc = qrows[b], krows[b], outc[b]

            def grp_body(g, carry2):
                base = g * K
                accs = [qr[base + a] * kr[base + a] for a in range(4)]
                for e in range(4, K, 4):
                    for a in range(4):
                        accs[a] = accs[a] + qr[base + e + a] * kr[base + e + a]
                oc[g] = ((accs[0] + accs[1]) + (accs[2] + accs[3])) * (1.0 / K)
                return carry2

            lax.fori_loop(0, CHUNK_G, grp_body, 0, unroll=False)
            return pltpu.async_copy(
                oc, out_hbm.at[pl.ds(wid * GPW + c * CHUNK_G, CHUNK_G)], osem[b])

        h_idx = [None] * NBUF
        h_g = [None] * NBUF
        h_out = [None] * NBUF
        for p in range(NBUF - 1):
            h_idx[p] = fire_idx(p, p)
            h_g[p] = fire_gather(p, p, h_idx[p])
        h_idx[NBUF - 1] = fire_idx(NBUF - 1, NBUF - 1)
        for c in range(NCHUNK):
            b = c % NBUF
            nb = (c + NBUF - 1) % NBUF
            if c + NBUF - 1 < NCHUNK:
                h_g[nb] = fire_gather(c + NBUF - 1, nb, h_idx[nb])
            for hh in h_g[b]:
                hh.wait()
            if c + NBUF < NCHUNK:
                h_idx[b] = fire_idx(c + NBUF, b)
            if h_out[b] is not None:
                h_out[b].wait()
            h_out[b] = compute(c, b)
        for hob in h_out:
            if hob is not None:
                hob.wait()

    return _sc_gather


def _out_body(qk_ref, sl_ref, wv_ref, b_ref, o_ref):
    q = qk_ref[:, :H]
    s = q * q + sl_ref[:N, :]
    norm = jnp.sum(s, axis=1, keepdims=True) + 1e-9
    attn = s / norm
    wvn = _nonneg(wv_ref[...])
    bn = _nonneg(b_ref[...])
    o_ref[...] = lax.dot_general(
        attn, wvn,
        dimension_numbers=(((1,), (1,)), ((), ())),
        preferred_element_type=jnp.float32,
    ) + bn


def _out_call(qk, sl_pad, Wv, b):
    return pl.pallas_call(
        _out_body,
        out_shape=jax.ShapeDtypeStruct((N, D), jnp.float32),
    )(qk, sl_pad, Wv, b)


def kernel(adj_list, x, Wq, Wk, Wv, b):
    w_cat = jnp.concatenate([Wq, Wk], axis=0)
    qk = _emb_call(x, w_cat)
    q_emb = qk[:, :H]
    k_emb = qk[:, H:]

    pad = EPAD - E
    a0 = jnp.concatenate([adj_list[0], jnp.zeros((pad,), adj_list.dtype)])
    a1 = jnp.concatenate([adj_list[1], jnp.zeros((pad,), adj_list.dtype)])

    sl_pad = _make_sc_gather()(q_emb, k_emb, a0, a1)

    return _out_call(qk, sl_pad, Wv, b)


# speedup vs baseline: 1.2104x; 1.2104x over previous
import functools
import math

import jax
import jax.numpy as jnp
from jax import lax
from jax.experimental import pallas as pl
from jax.experimental.pallas import tpu as pltpu
from jax.experimental.pallas import tpu_sc as plsc

N = 10000
E = 320000
D = 128
H = 16
K = 32

NC = 2
NS = 16
NW = NC * NS

GPAD = 10240
EPAD = GPAD * K
GPW = GPAD // NW
CHUNK_G = 32
CHUNK_E = CHUNK_G * K
NCHUNK = GPW // CHUNK_G
BLK = CHUNK_E // 128
BPW = EPAD // 128 // NW
NBUF = 3
NSPLIT = 4

_INV_SCALE = 1.0 / math.sqrt(float(D))


def _nonneg(w):
    return jnp.where(w > 0, w + 1.0, jnp.exp(w))


def _emb_body(x_ref, w_ref, o_ref):
    wn = _nonneg(w_ref[...])
    o_ref[...] = lax.dot_general(
        x_ref[...], wn,
        dimension_numbers=(((1,), (1,)), ((), ())),
        preferred_element_type=jnp.float32,
    ) * _INV_SCALE


def _emb_call(x, w_cat):
    return pl.pallas_call(
        _emb_body,
        out_shape=jax.ShapeDtypeStruct((N, 2 * H), jnp.float32),
    )(x, w_cat)


@functools.lru_cache(maxsize=None)
def _make_sc_gather():
    mesh = plsc.VectorSubcoreMesh(
        core_axis_name="c", subcore_axis_name="s", num_cores=NC, num_subcores=NS)

    @functools.partial(
        pl.kernel,
        out_type=jax.ShapeDtypeStruct((GPAD, H), jnp.float32),
        mesh=mesh,
        compiler_params=pltpu.CompilerParams(use_tc_tiling_on_sc=False),
        scratch_types=[
            [pltpu.VMEM((CHUNK_E,), jnp.int32)] * NBUF,
            [pltpu.VMEM((CHUNK_E,), jnp.int32)] * NBUF,
            [pltpu.VMEM((CHUNK_E, H), jnp.float32)] * NBUF,
            [pltpu.VMEM((CHUNK_E, H), jnp.float32)] * NBUF,
            [pltpu.VMEM((CHUNK_G, H), jnp.float32)] * NBUF,
            [pltpu.SemaphoreType.DMA] * NBUF,
            [pltpu.SemaphoreType.DMA] * NBUF,
            [pltpu.SemaphoreType.DMA] * NBUF,
            [pltpu.SemaphoreType.DMA] * NBUF,
            pltpu.VMEM_SHARED((N, H), jnp.float32),
            pltpu.VMEM_SHARED((N, H), jnp.float32),
            pltpu.SemaphoreType.DMA,
        ],
    )
    def _sc_gather(q_hbm, k_hbm, a0_hbm, a1_hbm, out_hbm,
                   idx_k, idx_q, krows, qrows, outc, gsem, gsem2, isem, osem,
                   q_sp, k_sp, ssem):
        wid = lax.axis_index("s") * NC + lax.axis_index("c")
        sid = lax.axis_index("s")

        rows_per_tile = N // NS
        r0 = sid * rows_per_tile
        h1 = pltpu.async_copy(q_hbm.at[pl.ds(r0, rows_per_tile)],
                              q_sp.at[pl.ds(r0, rows_per_tile)], ssem)
        h2 = pltpu.async_copy(k_hbm.at[pl.ds(r0, rows_per_tile)],
                              k_sp.at[pl.ds(r0, rows_per_tile)], ssem)
        h1.wait()
        h2.wait()
        plsc.subcore_barrier()

        def fire_idx(c, b):
            e0 = wid * (GPW * K) + c * CHUNK_E
            return [
                pltpu.async_copy(a0_hbm.at[pl.ds(e0, CHUNK_E)], idx_k[b], isem[b]),
                pltpu.async_copy(a1_hbm.at[pl.ds(e0, CHUNK_E)], idx_q[b], isem[b]),
            ]

        def fire_gather(c, b, idx_handles):
            for hh in idx_handles:
                hh.wait()
            handles = []
            for s in range(NSPLIT):
                sl = pl.ds(s * (CHUNK_E // NSPLIT), CHUNK_E // NSPLIT)
                handles.append(pltpu.async_copy(
                    k_sp.at[idx_k[b].at[sl]], krows[b].at[sl], gsem2[b]))
                handles.append(pltpu.async_copy(
                    q_sp.at[idx_q[b].at[sl]], qrows[b].at[sl], gsem[b]))
            return handles

        def compute(c, b):
            qr, kr, oc = qrows[b], krows[b], outc[b]

            def grp_body(g, carry2):
                base = g * K
                accs = [qr[base + a] * kr[base + a] for a in range(4)]
                for e in range(4, K, 4):
                    for a in range(4):
                        accs[a] = accs[a] + qr[base + e + a] * kr[base + e + a]
                oc[g] = ((accs[0] + accs[1]) + (accs[2] + accs[3])) * (1.0 / K)
                return carry2

            lax.fori_loop(0, CHUNK_G, grp_body, 0, unroll=False)
            return pltpu.async_copy(
                oc, out_hbm.at[pl.ds(wid * GPW + c * CHUNK_G, CHUNK_G)], osem[b])

        h_idx = [None] * NBUF
        h_g = [None] * NBUF
        h_out = [None] * NBUF
        for p in range(NBUF - 1):
            h_idx[p] = fire_idx(p, p)
            h_g[p] = fire_gather(p, p, h_idx[p])
        h_idx[NBUF - 1] = fire_idx(NBUF - 1, NBUF - 1)
        for c in range(NCHUNK):
            b = c % NBUF
            nb = (c + NBUF - 1) % NBUF
            if c + NBUF - 1 < NCHUNK:
                h_g[nb] = fire_gather(c + NBUF - 1, nb, h_idx[nb])
            for hh in h_g[b]:
                hh.wait()
            if c + NBUF < NCHUNK:
                h_idx[b] = fire_idx(c + NBUF, b)
            if h_out[b] is not None:
                h_out[b].wait()
            h_out[b] = compute(c, b)
        for hob in h_out:
            if hob is not None:
                hob.wait()

    return _sc_gather


def _out_body(qk_ref, sl_ref, wv_ref, b_ref, o_ref):
    q = qk_ref[:, :H]
    s = q * q + sl_ref[:N, :]
    norm = jnp.sum(s, axis=1, keepdims=True) + 1e-9
    attn = s / norm
    wvn = _nonneg(wv_ref[...])
    bn = _nonneg(b_ref[...])
    o_ref[...] = lax.dot_general(
        attn, wvn,
        dimension_numbers=(((1,), (1,)), ((), ())),
        preferred_element_type=jnp.float32,
    ) + bn


def _out_call(qk, sl_pad, Wv, b):
    return pl.pallas_call(
        _out_body,
        out_shape=jax.ShapeDtypeStruct((N, D), jnp.float32),
    )(qk, sl_pad, Wv, b)


def kernel(adj_list, x, Wq, Wk, Wv, b):
    w_cat = jnp.concatenate([Wq, Wk], axis=0)
    qk = _emb_call(x, w_cat)
    q_emb = qk[:, :H]
    k_emb = qk[:, H:]

    pad = EPAD - E
    a0 = jnp.concatenate([adj_list[0], jnp.zeros((pad,), adj_list.dtype)])
    a1 = jnp.concatenate([adj_list[1], jnp.zeros((pad,), adj_list.dtype)])

    sl_pad = _make_sc_gather()(q_emb, k_emb, a0, a1)

    return _out_call(qk, sl_pad, Wv, b)
